# baseline (speedup 1.0000x reference)
import functools

import jax
import jax.numpy as jnp
from jax import lax
from jax.experimental import pallas as pl
from jax.experimental.pallas import tpu as pltpu
from jax.experimental.pallas import tpu_sc as plsc

D_MODEL = 2048
N_EXPERTS = 64
TOP_K = 8
VOCAB = 129280
TOKENS = 8192
ROUTE_SCALE = 2.5


_NUM_CORES = 2
_NUM_SUBCORES = 16
_NW = _NUM_CORES * _NUM_SUBCORES
_TOK_PER_W = TOKENS // _NW
_FLAT_PER_W = _TOK_PER_W * TOP_K
_LANES = 16
_CHUNK = 128


def _sc_gather_indices(table_kmajor, input_ids):
    mesh = plsc.VectorSubcoreMesh(core_axis_name="c", subcore_axis_name="s")

    @functools.partial(
        pl.kernel,
        mesh=mesh,
        out_type=jax.ShapeDtypeStruct((TOP_K * TOKENS,), jnp.int32),
        scratch_types=[
            pltpu.VMEM((_TOK_PER_W,), jnp.int32),
            pltpu.VMEM((_FLAT_PER_W,), jnp.int32),
            pltpu.VMEM((_FLAT_PER_W,), jnp.int32),
            pltpu.SemaphoreType.DMA,
        ],
    )
    def gather_kernel(table_hbm, ids_hbm, out_hbm, ids_v, fidx_v, rows_v, sem):
        wid = lax.axis_index("s") * _NUM_CORES + lax.axis_index("c")
        base = wid * _TOK_PER_W
        pltpu.sync_copy(ids_hbm.at[pl.ds(base, _TOK_PER_W)], ids_v)

        def build(i, carry):
            v = ids_v[pl.ds(i * _LANES, _LANES)]
            for k in range(TOP_K):
                fidx_v[pl.ds(k * _TOK_PER_W + i * _LANES, _LANES)] = v + k * VOCAB
            return carry

        lax.fori_loop(0, _TOK_PER_W // _LANES, build, 0)

        copies = [
            pltpu.async_copy(
                table_hbm.at[fidx_v.at[pl.ds(j * _CHUNK, _CHUNK)]],
                rows_v.at[pl.ds(j * _CHUNK, _CHUNK)],
                sem,
            )
            for j in range(_FLAT_PER_W // _CHUNK)
        ]
        for c in copies:
            c.wait()
        for k in range(TOP_K):
            pltpu.sync_copy(
                rows_v.at[pl.ds(k * _TOK_PER_W, _TOK_PER_W)],
                out_hbm.at[pl.ds(k * TOKENS + base, _TOK_PER_W)],
            )

    return gather_kernel(table_kmajor, input_ids)


_BLK_T = 1024


def _tc_scores_body(x_ref, w_ref, out_ref):
    scores = lax.dot_general(
        x_ref[...], w_ref[...], (((1,), (1,)), ((), ())),
        preferred_element_type=jnp.float32,
    )
    sp = jnp.maximum(scores, 0.0) + jnp.log(1.0 + jnp.exp(-jnp.abs(scores)))
    out_ref[...] = jnp.sqrt(sp)


def _tc_scores(x, weight):
    return pl.pallas_call(
        _tc_scores_body,
        grid=(TOKENS // _BLK_T,),
        in_specs=[
            pl.BlockSpec((_BLK_T, D_MODEL), lambda i: (i, 0)),
            pl.BlockSpec((N_EXPERTS, D_MODEL), lambda i: (0, 0)),
        ],
        out_specs=pl.BlockSpec((_BLK_T, N_EXPERTS), lambda i: (i, 0)),
        out_shape=jax.ShapeDtypeStruct((TOKENS, N_EXPERTS), jnp.float32),
    )(x, weight)


_BLK_S = 8192


def _tc_select_body(g_ref, idx_ref, out_ref):
    g = g_ref[...]
    idx = idx_ref[...]
    wsel = jnp.take_along_axis(g, idx, axis=1)
    denom = jnp.sum(wsel, axis=1, keepdims=True) + 1e-20
    out_ref[...] = wsel * (ROUTE_SCALE / denom)


def _tc_select(g, indices):
    return pl.pallas_call(
        _tc_select_body,
        grid=(TOKENS // _BLK_S,),
        in_specs=[
            pl.BlockSpec((_BLK_S, N_EXPERTS), lambda i: (i, 0)),
            pl.BlockSpec((_BLK_S, TOP_K), lambda i: (i, 0)),
        ],
        out_specs=pl.BlockSpec((_BLK_S, TOP_K), lambda i: (i, 0)),
        out_shape=jax.ShapeDtypeStruct((TOKENS, TOP_K), jnp.float32),
    )(g, indices)


def kernel(x, weight, tid2eid, input_ids):
    table_kmajor = tid2eid.T.reshape(-1)
    idx_kmajor = _sc_gather_indices(table_kmajor, input_ids)
    indices = idx_kmajor.reshape(TOP_K, TOKENS).T
    g = _tc_scores(x, weight)
    weights = _tc_select(g, indices)
    return (weights.astype(x.dtype), indices)


# speedup vs baseline: 1.0416x; 1.0416x over previous
import functools

import jax
import jax.numpy as jnp
from jax import lax
from jax.experimental import pallas as pl
from jax.experimental.pallas import tpu as pltpu
from jax.experimental.pallas import tpu_sc as plsc

D_MODEL = 2048
N_EXPERTS = 64
TOP_K = 8
VOCAB = 129280
TOKENS = 8192
ROUTE_SCALE = 2.5


_NUM_CORES = 2
_NUM_SUBCORES = 16
_NW = _NUM_CORES * _NUM_SUBCORES
_TOK_PER_W = TOKENS // _NW
_FLAT_PER_W = _TOK_PER_W * TOP_K
_LANES = 16
_CHUNK = 128


def _sc_gather_indices(table_kmajor, input_ids):
    mesh = plsc.VectorSubcoreMesh(core_axis_name="c", subcore_axis_name="s")

    @functools.partial(
        pl.kernel,
        mesh=mesh,
        out_type=jax.ShapeDtypeStruct((TOP_K * TOKENS,), jnp.int32),
        scratch_types=[
            pltpu.VMEM((_TOK_PER_W,), jnp.int32),
            pltpu.VMEM((_FLAT_PER_W,), jnp.int32),
            pltpu.VMEM((_FLAT_PER_W,), jnp.int32),
            pltpu.SemaphoreType.DMA,
        ],
    )
    def gather_kernel(table_hbm, ids_hbm, out_hbm, ids_v, fidx_v, rows_v, sem):
        wid = lax.axis_index("s") * _NUM_CORES + lax.axis_index("c")
        base = wid * _TOK_PER_W
        pltpu.sync_copy(ids_hbm.at[pl.ds(base, _TOK_PER_W)], ids_v)

        def build(i, carry):
            v = ids_v[pl.ds(i * _LANES, _LANES)]
            for k in range(TOP_K):
                fidx_v[pl.ds(k * _TOK_PER_W + i * _LANES, _LANES)] = v + k * VOCAB
            return carry

        lax.fori_loop(0, _TOK_PER_W // _LANES, build, 0)

        copies = [
            pltpu.async_copy(
                table_hbm.at[fidx_v.at[pl.ds(j * _CHUNK, _CHUNK)]],
                rows_v.at[pl.ds(j * _CHUNK, _CHUNK)],
                sem,
            )
            for j in range(_FLAT_PER_W // _CHUNK)
        ]
        for c in copies:
            c.wait()
        for k in range(TOP_K):
            pltpu.sync_copy(
                rows_v.at[pl.ds(k * _TOK_PER_W, _TOK_PER_W)],
                out_hbm.at[pl.ds(k * TOKENS + base, _TOK_PER_W)],
            )

    return gather_kernel(table_kmajor, input_ids)


_BLK_T = 1024


def _tc_scores_body(x_ref, w_ref, out_ref):
    scores = lax.dot_general(
        x_ref[...], w_ref[...], (((1,), (1,)), ((), ())),
        preferred_element_type=jnp.float32,
    )
    sp = jnp.maximum(scores, 0.0) + jnp.log(1.0 + jnp.exp(-jnp.abs(scores)))
    out_ref[...] = jnp.sqrt(sp).astype(jnp.bfloat16)


def _tc_scores(x, weight):
    return pl.pallas_call(
        _tc_scores_body,
        grid=(TOKENS // _BLK_T,),
        in_specs=[
            pl.BlockSpec((_BLK_T, D_MODEL), lambda i: (i, 0)),
            pl.BlockSpec((N_EXPERTS, D_MODEL), lambda i: (0, 0)),
        ],
        out_specs=pl.BlockSpec((_BLK_T, N_EXPERTS), lambda i: (i, 0)),
        out_shape=jax.ShapeDtypeStruct((TOKENS, N_EXPERTS), jnp.bfloat16),
    )(x, weight)


_BLK_S = 4096


def _tc_select_body(g_ref, idx_ref, out_ref):
    g = g_ref[...]
    idx = idx_ref[...]
    wsel = jnp.take_along_axis(g.astype(jnp.float32), idx, axis=1)
    denom = jnp.sum(wsel, axis=1, keepdims=True) + 1e-20
    out_ref[...] = wsel * (ROUTE_SCALE / denom)


def _tc_select(g, indices):
    return pl.pallas_call(
        _tc_select_body,
        grid=(TOKENS // _BLK_S,),
        in_specs=[
            pl.BlockSpec((_BLK_S, N_EXPERTS), lambda i: (i, 0)),
            pl.BlockSpec((_BLK_S, TOP_K), lambda i: (i, 0)),
        ],
        out_specs=pl.BlockSpec((_BLK_S, TOP_K), lambda i: (i, 0)),
        out_shape=jax.ShapeDtypeStruct((TOKENS, TOP_K), jnp.float32),
    )(g, indices)


def kernel(x, weight, tid2eid, input_ids):
    table_kmajor = tid2eid.T.reshape(-1)
    idx_kmajor = _sc_gather_indices(table_kmajor, input_ids)
    indices = idx_kmajor.reshape(TOP_K, TOKENS).T
    g = _tc_scores(x, weight)
    weights = _tc_select(g, indices)
    return (weights.astype(x.dtype), indices)
